# baseline (speedup 1.0000x reference)
import jax
import jax.numpy as jnp
from jax import lax
from jax.experimental import pallas as pl
from jax.experimental.pallas import tpu as pltpu
from jax.experimental.pallas import tpu_sc as plsc

N = 262144
C3 = 128
CH = 3 * C3
NC = 2
NS = 16
NW = NC * NS
PER_W = N // NW
CHUNK = 128
NBUF = 2
N_CHUNKS = PER_W // CHUNK


def _sc_body(idx_hbm, tab_hbm, out_hbm, idx_v, rows_v, *sems):
    cid = lax.axis_index("c")
    sid = lax.axis_index("s")
    base0 = (sid * NC + cid) * (3 * PER_W)
    gsem = sems[:NBUF]
    wsem = sems[NBUF:]

    pltpu.sync_copy(idx_hbm.at[pl.ds(base0, 3 * PER_W)], idx_v)

    def fire_gathers(i, b):
        for p in range(3):
            off = i * (3 * CHUNK) + p * CHUNK
            pltpu.async_copy(tab_hbm.at[idx_v.at[pl.ds(off, CHUNK)]],
                             rows_v.at[b, pl.ds(p * CHUNK, CHUNK)], gsem[b])

    def wait_gathers(b):
        for p in range(3):
            pltpu.make_async_copy(tab_hbm.at[idx_v.at[pl.ds(0, CHUNK)]],
                                  rows_v.at[b, pl.ds(p * CHUNK, CHUNK)],
                                  gsem[b]).wait()

    def fire_write(i, b):
        base = base0 + i * (3 * CHUNK)
        pltpu.async_copy(rows_v.at[b], out_hbm.at[pl.ds(base, 3 * CHUNK)],
                         wsem[b])

    def wait_write(b):
        pltpu.make_async_copy(rows_v.at[b], out_hbm.at[pl.ds(base0, 3 * CHUNK)],
                              wsem[b]).wait()

    fire_gathers(0, 0)

    def group_body(g, carry):
        for b in range(NBUF):
            i = g * NBUF + b

            if b == 0:
                @pl.when(g >= 1)
                def _():
                    wait_write(0)
                    fire_gathers(i, 0)
                    wait_gathers(NBUF - 1)
                    fire_write(i - 1, NBUF - 1)
            else:
                @pl.when(g >= 1)
                def _():
                    wait_write(b)
                fire_gathers(i, b)
                wait_gathers(b - 1)
                fire_write(i - 1, b - 1)
        return carry

    lax.fori_loop(0, N_CHUNKS // NBUF, group_body, 0)

    last = NBUF - 1
    wait_gathers(last)
    fire_write(N_CHUNKS - 1, last)
    for b in range(NBUF):
        wait_write(b)


def kernel(coords, embed_x, embed_y, embed_z):
    idx = (coords + jnp.array([0, 1024, 2048], jnp.int32)).reshape(-1)
    tab = jnp.concatenate([embed_x, embed_y, embed_z], axis=0)
    mesh = plsc.VectorSubcoreMesh(core_axis_name="c", subcore_axis_name="s")
    run = pl.kernel(
        _sc_body,
        out_type=jax.ShapeDtypeStruct((3 * N, C3), jnp.float32),
        mesh=mesh,
        scratch_types=[
            pltpu.VMEM((3 * PER_W,), jnp.int32),
            pltpu.VMEM((NBUF, 3 * CHUNK, C3), jnp.float32),
        ] + [pltpu.SemaphoreType.DMA] * (2 * NBUF),
    )
    return run(idx, tab).reshape(N, CH)


# speedup vs baseline: 2.5536x; 2.5536x over previous
import jax
import jax.numpy as jnp
from jax import lax
from jax.experimental import pallas as pl
from jax.experimental.pallas import tpu as pltpu
from jax.experimental.pallas import tpu_sc as plsc

N = 262144
C3 = 128
CH = 3 * C3
NC = 2
NS = 16
NW = NC * NS
PER_W = N // NW
CHUNK = 128
NBUF = 2
N_CHUNKS = PER_W // CHUNK


def _sc_body(cx_hbm, cy_hbm, cz_hbm, ex_hbm, ey_hbm, ez_hbm, out_hbm,
             ix_v, iy_v, iz_v, rows_v, *sems):
    cid = lax.axis_index("c")
    sid = lax.axis_index("s")
    base0 = (sid * NC + cid) * PER_W
    gsem = sems[:NBUF]
    wsem = sems[NBUF:]
    tables = (ex_hbm, ey_hbm, ez_hbm)
    coords = (cx_hbm, cy_hbm, cz_hbm)
    idx_v = (ix_v, iy_v, iz_v)

    for d in range(3):
        pltpu.sync_copy(coords[d].at[pl.ds(base0, PER_W)], idx_v[d])

    def fire_gathers(i, b):
        for d in range(3):
            pltpu.async_copy(tables[d].at[idx_v[d].at[pl.ds(i * CHUNK, CHUNK)]],
                             rows_v.at[b, :, pl.ds(d * C3, C3)], gsem[b])

    def wait_gathers(b):
        for d in range(3):
            pltpu.make_async_copy(tables[d].at[idx_v[d].at[pl.ds(0, CHUNK)]],
                                  rows_v.at[b, :, pl.ds(d * C3, C3)],
                                  gsem[b]).wait()

    def fire_write(i, b):
        base = base0 + i * CHUNK
        pltpu.async_copy(rows_v.at[b], out_hbm.at[pl.ds(base, CHUNK)], wsem[b])

    def wait_write(b):
        pltpu.make_async_copy(rows_v.at[b], out_hbm.at[pl.ds(base0, CHUNK)],
                              wsem[b]).wait()

    fire_gathers(0, 0)

    def group_body(g, carry):
        for b in range(NBUF):
            i = g * NBUF + b

            if b == 0:
                @pl.when(g >= 1)
                def _():
                    wait_write(0)
                    fire_gathers(i, 0)
                    wait_gathers(NBUF - 1)
                    fire_write(i - 1, NBUF - 1)
            else:
                @pl.when(g >= 1)
                def _():
                    wait_write(b)
                fire_gathers(i, b)
                wait_gathers(b - 1)
                fire_write(i - 1, b - 1)
        return carry

    lax.fori_loop(0, N_CHUNKS // NBUF, group_body, 0)

    last = NBUF - 1
    wait_gathers(last)
    fire_write(N_CHUNKS - 1, last)
    for b in range(NBUF):
        wait_write(b)


def kernel(coords, embed_x, embed_y, embed_z):
    cx = coords[:, 0]
    cy = coords[:, 1]
    cz = coords[:, 2]
    mesh = plsc.VectorSubcoreMesh(core_axis_name="c", subcore_axis_name="s")
    run = pl.kernel(
        _sc_body,
        out_type=jax.ShapeDtypeStruct((N, CH), jnp.float32),
        mesh=mesh,
        scratch_types=[
            pltpu.VMEM((PER_W,), jnp.int32),
            pltpu.VMEM((PER_W,), jnp.int32),
            pltpu.VMEM((PER_W,), jnp.int32),
            pltpu.VMEM((NBUF, CHUNK, CH), jnp.float32),
        ] + [pltpu.SemaphoreType.DMA] * (2 * NBUF),
    )
    return run(cx, cy, cz, embed_x, embed_y, embed_z)
